# Initial kernel scaffold; baseline (speedup 1.0000x reference)
import jax
import jax.numpy as jnp
from jax.experimental import pallas as pl


def kernel(x, codebooks, frame_rate):
    raise NotImplementedError("write your pallas kernel here")


# speedup vs baseline: 1.6003x; 1.6003x over previous
import jax
import jax.numpy as jnp
import numpy as np
from jax.experimental import pallas as pl
from jax.experimental.pallas import tpu as pltpu

N_Q = 8
BINS = 1024
DIM = 128
B = 16
T = 2048
BM = 512


def _rvq_kernel(x_ref, cb_ref, quant_ref, codes_ref, loss_ref):
    @pl.when(pl.program_id(0) == 0)
    def _init():
        loss_ref[...] = jnp.zeros_like(loss_ref)

    r = x_ref[0]
    qsum = jnp.zeros_like(r)
    idx_rows = []
    for i in range(N_Q):
        cb = cb_ref[i]
        rnorm = jnp.sum(r * r, axis=0, keepdims=True)
        cnorm = jnp.sum(cb * cb, axis=1, keepdims=True)
        scores = jax.lax.dot_general(
            cb, r, (((1,), (0,)), ((), ())),
            preferred_element_type=jnp.float32)
        dist = rnorm - 2.0 * scores + cnorm
        idx = jnp.argmin(dist, axis=0).reshape(1, BM)
        onehot = (jax.lax.broadcasted_iota(jnp.int32, (BINS, BM), 0)
                  == idx).astype(jnp.float32)
        quant = jax.lax.dot_general(
            cb, onehot, (((0,), (0,)), ((), ())),
            preferred_element_type=jnp.float32,
            precision=jax.lax.Precision.HIGHEST)
        sqerr_sum = jnp.sum((quant - r) ** 2)
        loss_ref[i, :] = loss_ref[i, :] + sqerr_sum * (1.0 / DIM)
        idx_rows.append(idx)
        r = r - quant
        qsum = qsum + quant
    quant_ref[0] = qsum
    codes_ref[...] = jnp.concatenate(idx_rows, axis=0)


def kernel(x, codebooks, frame_rate):
    n_blk_t = T // BM
    grid = (B * n_blk_t,)

    quant, codes, loss = pl.pallas_call(
        _rvq_kernel,
        grid=grid,
        in_specs=[
            pl.BlockSpec((1, DIM, BM),
                         lambda p: (p // n_blk_t, 0, p % n_blk_t)),
            pl.BlockSpec((N_Q, BINS, DIM), lambda p: (0, 0, 0)),
        ],
        out_specs=[
            pl.BlockSpec((1, DIM, BM),
                         lambda p: (p // n_blk_t, 0, p % n_blk_t)),
            pl.BlockSpec((N_Q, BM), lambda p: (0, p)),
            pl.BlockSpec((N_Q, DIM), lambda p: (0, 0)),
        ],
        out_shape=[
            jax.ShapeDtypeStruct((B, DIM, T), jnp.float32),
            jax.ShapeDtypeStruct((N_Q, B * T), jnp.int32),
            jax.ShapeDtypeStruct((N_Q, DIM), jnp.float32),
        ],
    )(x, codebooks)

    codes = codes.reshape(N_Q, B, T)
    commit_loss = jnp.sum(loss, axis=1) / (B * T * DIM)
    penalty = jnp.mean(commit_loss)
    bw = jnp.asarray(N_Q * np.log2(BINS) * frame_rate, dtype=x.dtype)
    return quant, codes, bw, penalty


# speedup vs baseline: 2.3790x; 1.4866x over previous
import jax
import jax.numpy as jnp
import numpy as np
from jax.experimental import pallas as pl
from jax.experimental.pallas import tpu as pltpu

N_Q = 8
BINS = 1024
DIM = 128
B = 16
T = 2048
BM = 512


def _rvq_kernel(x_ref, cb_ref, quant_ref, codes_ref, loss_ref):
    @pl.when(pl.program_id(0) == 0)
    def _init():
        loss_ref[...] = jnp.zeros_like(loss_ref)

    r = x_ref[0]
    qsum = jnp.zeros_like(r)
    idx_rows = []
    for i in range(N_Q):
        cb = cb_ref[i]
        rnorm = jnp.sum(r * r, axis=0, keepdims=True)
        cnorm = jnp.sum(cb * cb, axis=1, keepdims=True)
        scores = jax.lax.dot_general(
            cb, r, (((1,), (0,)), ((), ())),
            preferred_element_type=jnp.float32)
        dist = rnorm - 2.0 * scores + cnorm
        idx = jnp.argmin(dist, axis=0).reshape(1, BM)
        onehot = (jax.lax.broadcasted_iota(jnp.int32, (BINS, BM), 0)
                  == idx).astype(jnp.bfloat16)
        cb_hi = cb.astype(jnp.bfloat16)
        r1 = cb - cb_hi.astype(jnp.float32)
        cb_mid = r1.astype(jnp.bfloat16)
        cb_lo = (r1 - cb_mid.astype(jnp.float32)).astype(jnp.bfloat16)
        dn = (((0,), (0,)), ((), ()))
        quant = (jax.lax.dot_general(cb_hi, onehot, dn,
                                     preferred_element_type=jnp.float32)
                 + jax.lax.dot_general(cb_mid, onehot, dn,
                                       preferred_element_type=jnp.float32)
                 + jax.lax.dot_general(cb_lo, onehot, dn,
                                       preferred_element_type=jnp.float32))
        sqerr_sum = jnp.sum((quant - r) ** 2)
        loss_ref[i, :] = loss_ref[i, :] + sqerr_sum * (1.0 / DIM)
        idx_rows.append(idx)
        r = r - quant
        qsum = qsum + quant
    quant_ref[0] = qsum
    codes_ref[...] = jnp.concatenate(idx_rows, axis=0)


def kernel(x, codebooks, frame_rate):
    n_blk_t = T // BM
    grid = (B * n_blk_t,)

    quant, codes, loss = pl.pallas_call(
        _rvq_kernel,
        grid=grid,
        in_specs=[
            pl.BlockSpec((1, DIM, BM),
                         lambda p: (p // n_blk_t, 0, p % n_blk_t)),
            pl.BlockSpec((N_Q, BINS, DIM), lambda p: (0, 0, 0)),
        ],
        out_specs=[
            pl.BlockSpec((1, DIM, BM),
                         lambda p: (p // n_blk_t, 0, p % n_blk_t)),
            pl.BlockSpec((N_Q, BM), lambda p: (0, p)),
            pl.BlockSpec((N_Q, DIM), lambda p: (0, 0)),
        ],
        out_shape=[
            jax.ShapeDtypeStruct((B, DIM, T), jnp.float32),
            jax.ShapeDtypeStruct((N_Q, B * T), jnp.int32),
            jax.ShapeDtypeStruct((N_Q, DIM), jnp.float32),
        ],
    )(x, codebooks)

    codes = codes.reshape(N_Q, B, T)
    commit_loss = jnp.sum(loss, axis=1) / (B * T * DIM)
    penalty = jnp.mean(commit_loss)
    bw = jnp.asarray(N_Q * np.log2(BINS) * frame_rate, dtype=x.dtype)
    return quant, codes, bw, penalty
